# baseline (speedup 1.0000x reference)
import functools

import jax
import jax.numpy as jnp
from jax import lax
from jax.experimental import pallas as pl
from jax.experimental.pallas import tpu as pltpu
from jax.experimental.pallas import tpu_sc as plsc

N = 10000
NPAD = 10240
E = 320000
D = 128
NC = 2
NS = 16
NW = NC * NS
CHUNK = 128
CPW = 80
CPW0 = 40
CPW1 = 2 * CPW - CPW0
GRP = 8
EPAD = NW * CPW * CHUNK
RPS = NPAD // NS

_MESH = plsc.VectorSubcoreMesh(
    core_axis_name="c", subcore_axis_name="s", num_cores=NC, num_subcores=NS)


def _sc_hist(idx_r, zeros, ones):
  @functools.partial(
      pl.kernel,
      out_type=jax.ShapeDtypeStruct((NC, NPAD, D), jnp.float32),
      mesh=_MESH,
      scratch_types=[
          pltpu.VMEM((CPW, CHUNK), jnp.int32),
          pltpu.VMEM((CHUNK, D), jnp.float32),
          pltpu.VMEM_SHARED((NPAD, D), jnp.float32),
      ],
  )
  def k(idx_hbm, zeros_hbm, ones_hbm, out_hbm, idx_v, ones_v, acc):
    cid = lax.axis_index("c")
    sid = lax.axis_index("s")
    wid = sid * NC + cid

    pltpu.sync_copy(zeros_hbm.at[pl.ds(sid * RPS, RPS)],
                    acc.at[pl.ds(sid * RPS, RPS)])
    pltpu.sync_copy(ones_hbm, ones_v)
    pltpu.sync_copy(idx_hbm.at[pl.ds(wid * CPW, CPW)], idx_v)
    plsc.subcore_barrier()

    @pl.loop(0, CPW)
    def _(j):
      pltpu.sync_copy(ones_v, acc.at[idx_v.at[j]], add=True)

    plsc.subcore_barrier()

    pltpu.sync_copy(acc.at[pl.ds(sid * RPS, RPS)],
                    out_hbm.at[cid].at[pl.ds(sid * RPS, RPS)])

  return k(idx_r, zeros, ones)


def _sc_aggregate(table, src_r, dst_r, zeros):
  @functools.partial(
      pl.kernel,
      out_type=jax.ShapeDtypeStruct((NC, NPAD, D), jnp.float32),
      mesh=_MESH,
      scratch_types=[
          pltpu.VMEM((GRP, CHUNK), jnp.int32),
          pltpu.VMEM((GRP, CHUNK), jnp.int32),
          pltpu.VMEM((2, CHUNK, D), jnp.float32),
          pltpu.VMEM_SHARED((NPAD, D), jnp.float32),
          pltpu.SemaphoreType.DMA,
          pltpu.SemaphoreType.DMA,
      ],
  )
  def k(tab_hbm, src_hbm, dst_hbm, zeros_hbm, out_hbm,
        sidx, didx, rows, acc, sem0, sem1):
    cid = lax.axis_index("c")
    sid = lax.axis_index("s")
    start = jnp.where(cid == 0, sid * CPW0, NS * CPW0 + sid * CPW1)
    ngrp = jnp.where(cid == 0, CPW0 // GRP, CPW1 // GRP)

    pltpu.sync_copy(zeros_hbm.at[pl.ds(sid * RPS, RPS)],
                    acc.at[pl.ds(sid * RPS, RPS)])
    plsc.subcore_barrier()

    SUB = 4
    SLEN = CHUNK // SUB

    def gather(idx_row, buf, sem):
      for s in range(SUB):
        pltpu.async_copy(tab_hbm.at[idx_row.at[pl.ds(s * SLEN, SLEN)]],
                         buf.at[pl.ds(s * SLEN, SLEN)], sem)

    def gather_wait(idx_row, buf, sem):
      for s in range(SUB):
        pltpu.make_async_copy(tab_hbm.at[idx_row.at[pl.ds(s * SLEN, SLEN)]],
                              buf.at[pl.ds(s * SLEN, SLEN)], sem).wait()

    @pl.loop(0, ngrp)
    def _(grp):
      pltpu.sync_copy(src_hbm.at[pl.ds(start + grp * GRP, GRP)], sidx)
      pltpu.sync_copy(dst_hbm.at[pl.ds(start + grp * GRP, GRP)], didx)

      gather(sidx.at[0], rows.at[0], sem0)

      @pl.loop(0, GRP - 2, step=2)
      def _(j):
        gather(sidx.at[j + 1], rows.at[1], sem1)
        gather_wait(sidx.at[j], rows.at[0], sem0)
        pltpu.sync_copy(rows.at[0], acc.at[didx.at[j]], add=True)
        gather(sidx.at[j + 2], rows.at[0], sem0)
        gather_wait(sidx.at[j + 1], rows.at[1], sem1)
        pltpu.sync_copy(rows.at[1], acc.at[didx.at[j + 1]], add=True)

      gather(sidx.at[GRP - 1], rows.at[1], sem1)
      gather_wait(sidx.at[GRP - 2], rows.at[0], sem0)
      pltpu.sync_copy(rows.at[0], acc.at[didx.at[GRP - 2]], add=True)
      gather_wait(sidx.at[GRP - 1], rows.at[1], sem1)
      pltpu.sync_copy(rows.at[1], acc.at[didx.at[GRP - 1]], add=True)

    plsc.subcore_barrier()

    pltpu.sync_copy(acc.at[pl.ds(sid * RPS, RPS)],
                    out_hbm.at[cid].at[pl.ds(sid * RPS, RPS)])

  return k(table, src_r, dst_r, zeros)


_BLK = 1024


def _norm(part_ref, blk):
  d = part_ref[0, :, 0] + part_ref[1, :, 0]
  return lax.rsqrt(jnp.maximum(d, 1.0)).reshape(blk, 1)


def _tc_mm_scale(x, w, h0):
  def body(x_ref, w_ref, h0_ref, o_ref):
    o_ref[...] = jnp.dot(x_ref[...], w_ref[...],
                         preferred_element_type=jnp.float32) * _norm(h0_ref,
                                                                     _BLK)

  return pl.pallas_call(
      body,
      grid=(NPAD // _BLK,),
      in_specs=[
          pl.BlockSpec((_BLK, D), lambda i: (i, 0)),
          pl.BlockSpec((D, D), lambda i: (0, 0)),
          pl.BlockSpec((NC, _BLK, D), lambda i: (0, i, 0)),
      ],
      out_specs=pl.BlockSpec((_BLK, D), lambda i: (i, 0)),
      out_shape=jax.ShapeDtypeStruct((NPAD, D), jnp.float32),
  )(x, w, h0)


def _tc_mid(p, h0, h1, w2):
  def body(p_ref, h0_ref, h1_ref, w_ref, o_ref):
    h = jnp.maximum((p_ref[0] + p_ref[1]) * _norm(h1_ref, _BLK), 0.0)
    o_ref[...] = jnp.dot(h, w_ref[...],
                         preferred_element_type=jnp.float32) * _norm(h0_ref,
                                                                     _BLK)

  return pl.pallas_call(
      body,
      grid=(NPAD // _BLK,),
      in_specs=[
          pl.BlockSpec((NC, _BLK, D), lambda i: (0, i, 0)),
          pl.BlockSpec((NC, _BLK, D), lambda i: (0, i, 0)),
          pl.BlockSpec((NC, _BLK, D), lambda i: (0, i, 0)),
          pl.BlockSpec((D, D), lambda i: (0, 0)),
      ],
      out_specs=pl.BlockSpec((_BLK, D), lambda i: (i, 0)),
      out_shape=jax.ShapeDtypeStruct((NPAD, D), jnp.float32),
  )(p, h0, h1, w2)


def _tc_out(q, h1):
  def body(q_ref, h1_ref, o_ref):
    o_ref[...] = (q_ref[0] + q_ref[1]) * _norm(h1_ref, _BLK)

  return pl.pallas_call(
      body,
      grid=(NPAD // _BLK,),
      in_specs=[
          pl.BlockSpec((NC, _BLK, D), lambda i: (0, i, 0)),
          pl.BlockSpec((NC, _BLK, D), lambda i: (0, i, 0)),
      ],
      out_specs=pl.BlockSpec((_BLK, D), lambda i: (i, 0)),
      out_shape=jax.ShapeDtypeStruct((NPAD, D), jnp.float32),
  )(q, h1)


def kernel(g, features, W1, W2):
  src = g[0].astype(jnp.int32)
  dst = g[1].astype(jnp.int32)
  pad = EPAD - E
  fill = jnp.full((pad,), N, jnp.int32)
  src_r = jnp.concatenate([src, fill]).reshape(NW * CPW, CHUNK)
  dst_r = jnp.concatenate([dst, fill]).reshape(NW * CPW, CHUNK)
  xp = jnp.pad(features, ((0, NPAD - N), (0, 0)))
  zeros = jnp.zeros((NPAD, D), jnp.float32)
  ones = jnp.ones((CHUNK, D), jnp.float32)

  h0 = _sc_hist(src_r, zeros, ones)
  h1 = _sc_hist(dst_r, zeros, ones)
  t1 = _tc_mm_scale(xp, W1, h0)
  p = _sc_aggregate(t1, src_r, dst_r, zeros)
  t2 = _tc_mid(p, h0, h1, W2)
  q = _sc_aggregate(t2, src_r, dst_r, zeros)
  out = _tc_out(q, h1)
  return out[:N]


# speedup vs baseline: 1.4001x; 1.4001x over previous
import functools

import jax
import jax.numpy as jnp
from jax import lax
from jax.experimental import pallas as pl
from jax.experimental.pallas import tpu as pltpu
from jax.experimental.pallas import tpu_sc as plsc

N = 10000
NPAD = 10240
E = 320000
D = 128
NC = 2
NS = 16
NW = NC * NS
CHUNK = 128
CPW = 80
GRP = 16
EPAD = NW * CPW * CHUNK
RPS = NPAD // NS

_MESH = plsc.VectorSubcoreMesh(
    core_axis_name="c", subcore_axis_name="s", num_cores=NC, num_subcores=NS)


def _sc_hist(idx_r, zeros, ones):
  @functools.partial(
      pl.kernel,
      out_type=jax.ShapeDtypeStruct((NC, NPAD, D), jnp.float32),
      mesh=_MESH,
      scratch_types=[
          pltpu.VMEM((CPW, CHUNK), jnp.int32),
          pltpu.VMEM((CHUNK, D), jnp.float32),
          pltpu.VMEM_SHARED((NPAD, D), jnp.float32),
      ],
  )
  def k(idx_hbm, zeros_hbm, ones_hbm, out_hbm, idx_v, ones_v, acc):
    cid = lax.axis_index("c")
    sid = lax.axis_index("s")
    wid = sid * NC + cid

    pltpu.sync_copy(zeros_hbm.at[pl.ds(sid * RPS, RPS)],
                    acc.at[pl.ds(sid * RPS, RPS)])
    pltpu.sync_copy(ones_hbm, ones_v)
    pltpu.sync_copy(idx_hbm.at[wid], idx_v)
    plsc.subcore_barrier()

    @pl.loop(0, CPW)
    def _(j):
      pltpu.sync_copy(ones_v, acc.at[idx_v.at[j]], add=True)

    plsc.subcore_barrier()

    pltpu.sync_copy(acc.at[pl.ds(sid * RPS, RPS)],
                    out_hbm.at[cid].at[pl.ds(sid * RPS, RPS)])

  return k(idx_r, zeros, ones)


def _sc_aggregate(table, src_r, dst_r, zeros):
  @functools.partial(
      pl.kernel,
      out_type=jax.ShapeDtypeStruct((NC, NPAD, D), jnp.float32),
      mesh=_MESH,
      scratch_types=[
          pltpu.VMEM((GRP, CHUNK), jnp.int32),
          pltpu.VMEM((GRP, CHUNK), jnp.int32),
          pltpu.VMEM((2, CHUNK, D), jnp.float32),
          pltpu.VMEM_SHARED((NPAD, D), jnp.float32),
          pltpu.SemaphoreType.DMA,
          pltpu.SemaphoreType.DMA,
      ],
  )
  def k(tab_hbm, src_hbm, dst_hbm, zeros_hbm, out_hbm,
        sidx, didx, rows, acc, sem0, sem1):
    cid = lax.axis_index("c")
    sid = lax.axis_index("s")
    wid = sid * NC + cid

    pltpu.sync_copy(zeros_hbm.at[pl.ds(sid * RPS, RPS)],
                    acc.at[pl.ds(sid * RPS, RPS)])
    plsc.subcore_barrier()

    SUB = 4
    SLEN = CHUNK // SUB

    def gather(idx_row, buf, sem):
      for s in range(SUB):
        pltpu.async_copy(tab_hbm.at[idx_row.at[pl.ds(s * SLEN, SLEN)]],
                         buf.at[pl.ds(s * SLEN, SLEN)], sem)

    def gather_wait(idx_row, buf, sem):
      for s in range(SUB):
        pltpu.make_async_copy(tab_hbm.at[idx_row.at[pl.ds(s * SLEN, SLEN)]],
                              buf.at[pl.ds(s * SLEN, SLEN)], sem).wait()

    @pl.loop(0, CPW // GRP)
    def _(grp):
      pltpu.sync_copy(src_hbm.at[wid].at[pl.ds(grp * GRP, GRP)], sidx)
      pltpu.sync_copy(dst_hbm.at[wid].at[pl.ds(grp * GRP, GRP)], didx)

      gather(sidx.at[0], rows.at[0], sem0)

      @pl.loop(0, GRP - 2, step=2)
      def _(j):
        gather(sidx.at[j + 1], rows.at[1], sem1)
        gather_wait(sidx.at[j], rows.at[0], sem0)
        pltpu.sync_copy(rows.at[0], acc.at[didx.at[j]], add=True)
        gather(sidx.at[j + 2], rows.at[0], sem0)
        gather_wait(sidx.at[j + 1], rows.at[1], sem1)
        pltpu.sync_copy(rows.at[1], acc.at[didx.at[j + 1]], add=True)

      gather(sidx.at[GRP - 1], rows.at[1], sem1)
      gather_wait(sidx.at[GRP - 2], rows.at[0], sem0)
      pltpu.sync_copy(rows.at[0], acc.at[didx.at[GRP - 2]], add=True)
      gather_wait(sidx.at[GRP - 1], rows.at[1], sem1)
      pltpu.sync_copy(rows.at[1], acc.at[didx.at[GRP - 1]], add=True)

    plsc.subcore_barrier()

    pltpu.sync_copy(acc.at[pl.ds(sid * RPS, RPS)],
                    out_hbm.at[cid].at[pl.ds(sid * RPS, RPS)])

  return k(table, src_r, dst_r, zeros)


_BLK = 1024


def _norm(part_ref, blk):
  d = part_ref[0, :, 0] + part_ref[1, :, 0]
  return lax.rsqrt(jnp.maximum(d, 1.0)).reshape(blk, 1)


def _tc_mm_scale(x, w, h0):
  def body(x_ref, w_ref, h0_ref, o_ref):
    o_ref[...] = jnp.dot(x_ref[...], w_ref[...],
                         preferred_element_type=jnp.float32) * _norm(h0_ref,
                                                                     _BLK)

  return pl.pallas_call(
      body,
      grid=(NPAD // _BLK,),
      in_specs=[
          pl.BlockSpec((_BLK, D), lambda i: (i, 0)),
          pl.BlockSpec((D, D), lambda i: (0, 0)),
          pl.BlockSpec((NC, _BLK, D), lambda i: (0, i, 0)),
      ],
      out_specs=pl.BlockSpec((_BLK, D), lambda i: (i, 0)),
      out_shape=jax.ShapeDtypeStruct((NPAD, D), jnp.float32),
  )(x, w, h0)


def _tc_mid(p, h0, h1, w2):
  def body(p_ref, h0_ref, h1_ref, w_ref, o_ref):
    h = jnp.maximum((p_ref[0] + p_ref[1]) * _norm(h1_ref, _BLK), 0.0)
    o_ref[...] = jnp.dot(h, w_ref[...],
                         preferred_element_type=jnp.float32) * _norm(h0_ref,
                                                                     _BLK)

  return pl.pallas_call(
      body,
      grid=(NPAD // _BLK,),
      in_specs=[
          pl.BlockSpec((NC, _BLK, D), lambda i: (0, i, 0)),
          pl.BlockSpec((NC, _BLK, D), lambda i: (0, i, 0)),
          pl.BlockSpec((NC, _BLK, D), lambda i: (0, i, 0)),
          pl.BlockSpec((D, D), lambda i: (0, 0)),
      ],
      out_specs=pl.BlockSpec((_BLK, D), lambda i: (i, 0)),
      out_shape=jax.ShapeDtypeStruct((NPAD, D), jnp.float32),
  )(p, h0, h1, w2)


def _tc_out(q, h1):
  def body(q_ref, h1_ref, o_ref):
    o_ref[...] = (q_ref[0] + q_ref[1]) * _norm(h1_ref, _BLK)

  return pl.pallas_call(
      body,
      grid=(NPAD // _BLK,),
      in_specs=[
          pl.BlockSpec((NC, _BLK, D), lambda i: (0, i, 0)),
          pl.BlockSpec((NC, _BLK, D), lambda i: (0, i, 0)),
      ],
      out_specs=pl.BlockSpec((_BLK, D), lambda i: (i, 0)),
      out_shape=jax.ShapeDtypeStruct((NPAD, D), jnp.float32),
  )(q, h1)


def kernel(g, features, W1, W2):
  src = g[0].astype(jnp.int32)
  dst = g[1].astype(jnp.int32)
  pad = EPAD - E
  fill = jnp.full((pad,), N, jnp.int32)
  src_r = jnp.concatenate([src, fill]).reshape(NW, CPW, CHUNK)
  dst_r = jnp.concatenate([dst, fill]).reshape(NW, CPW, CHUNK)
  xp = jnp.pad(features, ((0, NPAD - N), (0, 0)))
  zeros = jnp.zeros((NPAD, D), jnp.float32)
  ones = jnp.ones((CHUNK, D), jnp.float32)

  h0 = _sc_hist(src_r, zeros, ones)
  h1 = _sc_hist(dst_r, zeros, ones)
  t1 = _tc_mm_scale(xp, W1, h0)
  p = _sc_aggregate(t1, src_r, dst_r, zeros)
  t2 = _tc_mid(p, h0, h1, W2)
  q = _sc_aggregate(t2, src_r, dst_r, zeros)
  out = _tc_out(q, h1)
  return out[:N]
